# baseline (speedup 1.0000x reference)
import jax
import jax.numpy as jnp
from jax import lax
from jax.experimental import pallas as pl
from jax.experimental.pallas import tpu as pltpu
from jax.experimental.pallas import tpu_sc as plsc

N = 10000
D = 128
G = 256
NC, NS = 2, 16
NW = NC * NS
K = 128
NPAD = 10112
RPT = NPAD // NS
ZCH = (RPT - 1) // K + 1
BM = 1000
GRID = N // BM

_MESH = dict(core_axis_name="c", subcore_axis_name="s", num_cores=NC,
             num_subcores=NS)


def _agg_body(y_hbm, src_hbm, dst_hbm, zrows_hbm, outp_hbm, sidx_a, sidx_b,
              didx_a, didx_b, rows_a, rows_b, acc_sh, sem_a, sem_b):
    c = lax.axis_index("c")
    s = lax.axis_index("s")
    wid = c * NS + s
    r0 = s * RPT
    cpw = src_hbm.shape[1]
    npairs = cpw // 2
    pltpu.sync_copy(zrows_hbm, rows_a)
    for t in range(ZCH):
        n = min(K, RPT - t * K)
        pltpu.sync_copy(rows_a.at[pl.ds(0, n)],
                        acc_sh.at[pl.ds(r0 + t * K, n)])
    plsc.subcore_barrier()
    def pair(p, carry):
        j = 2 * p
        pltpu.sync_copy(src_hbm.at[wid, j], sidx_a)
        pltpu.async_copy(y_hbm.at[sidx_a], rows_a, sem_a)
        pltpu.sync_copy(dst_hbm.at[wid, j], didx_a)

        @pl.when(p > 0)
        def _():
            pltpu.make_async_copy(rows_b, acc_sh.at[didx_b], sem_b).wait()
        pltpu.make_async_copy(y_hbm.at[sidx_a], rows_a, sem_a).wait()
        pltpu.async_copy(rows_a, acc_sh.at[didx_a], sem_a, add=True)

        pltpu.sync_copy(src_hbm.at[wid, j + 1], sidx_b)
        pltpu.async_copy(y_hbm.at[sidx_b], rows_b, sem_b)
        pltpu.sync_copy(dst_hbm.at[wid, j + 1], didx_b)
        pltpu.make_async_copy(rows_a, acc_sh.at[didx_a], sem_a).wait()
        pltpu.make_async_copy(y_hbm.at[sidx_b], rows_b, sem_b).wait()
        pltpu.async_copy(rows_b, acc_sh.at[didx_b], sem_b, add=True)
        return carry

    lax.fori_loop(0, npairs, pair, 0)
    pltpu.make_async_copy(rows_b, acc_sh.at[didx_b], sem_b).wait()
    plsc.subcore_barrier()
    for t in range(ZCH):
        n = min(K, RPT - t * K)
        pltpu.sync_copy(acc_sh.at[pl.ds(r0 + t * K, n)],
                        rows_a.at[pl.ds(0, n)])
        pltpu.sync_copy(rows_a.at[pl.ds(0, n)],
                        outp_hbm.at[c, pl.ds(r0 + t * K, n)])


def _make_agg(cpw):
    return pl.kernel(
        _agg_body,
        out_type=jax.ShapeDtypeStruct((NC, NPAD, D), jnp.float32),
        mesh=plsc.VectorSubcoreMesh(**_MESH),
        scratch_types=[
            pltpu.VMEM((K,), jnp.int32),
            pltpu.VMEM((K,), jnp.int32),
            pltpu.VMEM((K,), jnp.int32),
            pltpu.VMEM((K,), jnp.int32),
            pltpu.VMEM((K, D), jnp.float32),
            pltpu.VMEM((K, D), jnp.float32),
            pltpu.VMEM_SHARED((NPAD, D), jnp.float32),
            pltpu.SemaphoreType.DMA,
            pltpu.SemaphoreType.DMA,
        ],
    )


def _dinv(degp_ref):
    dg = degp_ref[0, :, 0:1] + degp_ref[1, :, 0:1] + 1.0
    return lax.rsqrt(dg)


def _mm1_body(x_ref, w_ref, degp_ref, y_ref):
    y_ref[...] = _dinv(degp_ref) * jnp.dot(
        x_ref[...], w_ref[...], preferred_element_type=jnp.float32)


def _ep_body(accp_ref, yprev_ref, b_ref, degp_ref, w_ref, ynext_ref):
    dinv = _dinv(degp_ref)
    s = accp_ref[0] + accp_ref[1]
    h = jnp.maximum(dinv * (s + yprev_ref[...]) + b_ref[...], 0.0)
    ynext_ref[...] = dinv * jnp.dot(
        h, w_ref[...], preferred_element_type=jnp.float32)


def _pool_body(accp_ref, y_ref, b_ref, degp_ref, batch_ref, out_ref, sums,
               cnts):
    i = pl.program_id(0)
    dinv = _dinv(degp_ref)
    s = accp_ref[0] + accp_ref[1]
    h = jnp.maximum(dinv * (s + y_ref[...]) + b_ref[...], 0.0)
    bvec = batch_ref[0]
    gid = lax.broadcasted_iota(jnp.int32, (G, BM), 0)
    oh = (bvec == gid).astype(jnp.float32)

    @pl.when(i == 0)
    def _():
        sums[...] = jnp.zeros_like(sums)
        cnts[...] = jnp.zeros_like(cnts)

    sums[...] += jnp.dot(oh, h, preferred_element_type=jnp.float32)
    cnts[...] = cnts[...] + jnp.sum(oh, axis=1, keepdims=True)

    @pl.when(i == GRID - 1)
    def _():
        out_ref[...] = sums[...] / jnp.maximum(cnts[...], 1.0)


_spec_rows = pl.BlockSpec((BM, D), lambda i: (i, 0))
_spec_w = pl.BlockSpec((D, D), lambda i: (0, 0))
_spec_b = pl.BlockSpec((1, D), lambda i: (0, 0))
_spec_degp = pl.BlockSpec((NC, BM, D), lambda i: (0, i, 0))
_spec_accp = pl.BlockSpec((NC, BM, D), lambda i: (0, i, 0))

_mm1 = pl.pallas_call(
    _mm1_body,
    grid=(GRID,),
    in_specs=[_spec_rows, _spec_w, _spec_degp],
    out_specs=_spec_rows,
    out_shape=jax.ShapeDtypeStruct((N, D), jnp.float32),
)

_ep = pl.pallas_call(
    _ep_body,
    grid=(GRID,),
    in_specs=[_spec_accp, _spec_rows, _spec_b, _spec_degp, _spec_w],
    out_specs=_spec_rows,
    out_shape=jax.ShapeDtypeStruct((N, D), jnp.float32),
)

_pool = pl.pallas_call(
    _pool_body,
    grid=(GRID,),
    in_specs=[_spec_accp, _spec_rows, _spec_b, _spec_degp,
              pl.BlockSpec((1, 1, BM), lambda i: (i, 0, 0))],
    out_specs=pl.BlockSpec((G, D), lambda i: (0, 0)),
    out_shape=jax.ShapeDtypeStruct((G, D), jnp.float32),
    scratch_shapes=[pltpu.VMEM((G, D), jnp.float32),
                    pltpu.VMEM((G, D), jnp.float32)],
)


def kernel(x, edge_index, batch, W1, b1, W2, b2, W3, b3, W4, b4):
    src = edge_index[0].astype(jnp.int32)
    dst = edge_index[1].astype(jnp.int32)
    e = src.shape[0]
    cpw = 2 * (-(-e // (NW * K * 2)))
    epad = NW * K * cpw
    src3 = jnp.pad(src, (0, epad - e)).reshape(NW, cpw, K)
    dst3 = jnp.pad(dst, (0, epad - e),
                   constant_values=N).reshape(NW, cpw, K)
    ones_tbl = jnp.ones((NPAD, D), jnp.float32)
    zrows = jnp.zeros((K, D), jnp.float32)
    batch3 = batch.astype(jnp.int32).reshape(GRID, 1, BM)

    agg_k = _make_agg(cpw)

    degp = agg_k(ones_tbl, dst3, dst3, zrows)
    b1r, b2r, b3r, b4r = (b.reshape(1, D) for b in (b1, b2, b3, b4))

    y = _mm1(x, W1, degp)
    accp = agg_k(y, src3, dst3, zrows)
    y = _ep(accp, y, b1r, degp, W2)
    accp = agg_k(y, src3, dst3, zrows)
    y = _ep(accp, y, b2r, degp, W3)
    accp = agg_k(y, src3, dst3, zrows)
    y = _ep(accp, y, b3r, degp, W4)
    accp = agg_k(y, src3, dst3, zrows)
    return _pool(accp, y, b4r, degp, batch3)


# speedup vs baseline: 1.2992x; 1.2992x over previous
import jax
import jax.numpy as jnp
from jax import lax
from jax.experimental import pallas as pl
from jax.experimental.pallas import tpu as pltpu
from jax.experimental.pallas import tpu_sc as plsc

N = 10000
D = 128
G = 256
NC, NS = 2, 16
NW = NC * NS
K = 128
NPAD = 10112
RPT = NPAD // NS
ZCH = (RPT - 1) // K + 1
BM = 1000
GRID = N // BM

_MESH = dict(core_axis_name="c", subcore_axis_name="s", num_cores=NC,
             num_subcores=NS)


def _agg_body(y_hbm, src_hbm, dst_hbm, zrows_hbm, outp_hbm, sidx_v, didx_v,
              rows_v, acc_sh, sem):
    c = lax.axis_index("c")
    s = lax.axis_index("s")
    wid = c * NS + s
    r0 = s * RPT
    cpw = src_hbm.shape[1]
    pltpu.sync_copy(zrows_hbm, rows_v)
    for t in range(ZCH):
        n = min(K, RPT - t * K)
        pltpu.sync_copy(rows_v.at[pl.ds(0, n)],
                        acc_sh.at[pl.ds(r0 + t * K, n)])
    plsc.subcore_barrier()

    def step(j, carry):
        pltpu.sync_copy(src_hbm.at[wid, j], sidx_v)
        pltpu.async_copy(y_hbm.at[sidx_v], rows_v, sem).wait()
        pltpu.sync_copy(dst_hbm.at[wid, j], didx_v)
        pltpu.sync_copy(rows_v, acc_sh.at[didx_v], add=True)
        return carry

    lax.fori_loop(0, cpw, step, 0)
    plsc.subcore_barrier()
    for t in range(ZCH):
        n = min(K, RPT - t * K)
        pltpu.sync_copy(acc_sh.at[pl.ds(r0 + t * K, n)],
                        rows_v.at[pl.ds(0, n)])
        pltpu.sync_copy(rows_v.at[pl.ds(0, n)],
                        outp_hbm.at[c, pl.ds(r0 + t * K, n)])


def _make_agg(cpw):
    return pl.kernel(
        _agg_body,
        out_type=jax.ShapeDtypeStruct((NC, NPAD, D), jnp.float32),
        mesh=plsc.VectorSubcoreMesh(**_MESH),
        scratch_types=[
            pltpu.VMEM((K,), jnp.int32),
            pltpu.VMEM((K,), jnp.int32),
            pltpu.VMEM((K, D), jnp.float32),
            pltpu.VMEM_SHARED((NPAD, D), jnp.float32),
            pltpu.SemaphoreType.DMA,
        ],
    )


def _dinv(degp_ref):
    dg = degp_ref[0, :, 0:1] + degp_ref[1, :, 0:1] + 1.0
    return lax.rsqrt(dg)


def _mm1_body(x_ref, w_ref, degp_ref, y_ref):
    y_ref[...] = _dinv(degp_ref) * jnp.dot(
        x_ref[...], w_ref[...], preferred_element_type=jnp.float32)


def _ep_body(accp_ref, yprev_ref, b_ref, degp_ref, w_ref, ynext_ref):
    dinv = _dinv(degp_ref)
    s = accp_ref[0] + accp_ref[1]
    h = jnp.maximum(dinv * (s + yprev_ref[...]) + b_ref[...], 0.0)
    ynext_ref[...] = dinv * jnp.dot(
        h, w_ref[...], preferred_element_type=jnp.float32)


def _pool_body(accp_ref, y_ref, b_ref, degp_ref, batch_ref, out_ref, sums,
               cnts):
    i = pl.program_id(0)
    dinv = _dinv(degp_ref)
    s = accp_ref[0] + accp_ref[1]
    h = jnp.maximum(dinv * (s + y_ref[...]) + b_ref[...], 0.0)
    bvec = batch_ref[0]
    gid = lax.broadcasted_iota(jnp.int32, (G, BM), 0)
    oh = (bvec == gid).astype(jnp.float32)

    @pl.when(i == 0)
    def _():
        sums[...] = jnp.zeros_like(sums)
        cnts[...] = jnp.zeros_like(cnts)

    sums[...] += jnp.dot(oh, h, preferred_element_type=jnp.float32)
    cnts[...] = cnts[...] + jnp.sum(oh, axis=1, keepdims=True)

    @pl.when(i == GRID - 1)
    def _():
        out_ref[...] = sums[...] / jnp.maximum(cnts[...], 1.0)


_spec_rows = pl.BlockSpec((BM, D), lambda i: (i, 0))
_spec_w = pl.BlockSpec((D, D), lambda i: (0, 0))
_spec_b = pl.BlockSpec((1, D), lambda i: (0, 0))
_spec_degp = pl.BlockSpec((NC, BM, D), lambda i: (0, i, 0))
_spec_accp = pl.BlockSpec((NC, BM, D), lambda i: (0, i, 0))

_mm1 = pl.pallas_call(
    _mm1_body,
    grid=(GRID,),
    in_specs=[_spec_rows, _spec_w, _spec_degp],
    out_specs=_spec_rows,
    out_shape=jax.ShapeDtypeStruct((N, D), jnp.float32),
)

_ep = pl.pallas_call(
    _ep_body,
    grid=(GRID,),
    in_specs=[_spec_accp, _spec_rows, _spec_b, _spec_degp, _spec_w],
    out_specs=_spec_rows,
    out_shape=jax.ShapeDtypeStruct((N, D), jnp.float32),
)

_pool = pl.pallas_call(
    _pool_body,
    grid=(GRID,),
    in_specs=[_spec_accp, _spec_rows, _spec_b, _spec_degp,
              pl.BlockSpec((1, 1, BM), lambda i: (i, 0, 0))],
    out_specs=pl.BlockSpec((G, D), lambda i: (0, 0)),
    out_shape=jax.ShapeDtypeStruct((G, D), jnp.float32),
    scratch_shapes=[pltpu.VMEM((G, D), jnp.float32),
                    pltpu.VMEM((G, D), jnp.float32)],
)


def kernel(x, edge_index, batch, W1, b1, W2, b2, W3, b3, W4, b4):
    src = edge_index[0].astype(jnp.int32)
    dst = edge_index[1].astype(jnp.int32)
    e = src.shape[0]
    cpw = -(-e // (NW * K))
    epad = NW * K * cpw
    src3 = jnp.pad(src, (0, epad - e)).reshape(NW, cpw, K)
    dst3 = jnp.pad(dst, (0, epad - e),
                   constant_values=N).reshape(NW, cpw, K)
    ones_tbl = jnp.ones((NPAD, D), jnp.float32)
    zrows = jnp.zeros((K, D), jnp.float32)
    batch3 = batch.astype(jnp.int32).reshape(GRID, 1, BM)

    agg_k = _make_agg(cpw)

    degp = agg_k(ones_tbl, dst3, dst3, zrows)
    b1r, b2r, b3r, b4r = (b.reshape(1, D) for b in (b1, b2, b3, b4))

    y = _mm1(x, W1, degp)
    accp = agg_k(y, src3, dst3, zrows)
    y = _ep(accp, y, b1r, degp, W2)
    accp = agg_k(y, src3, dst3, zrows)
    y = _ep(accp, y, b2r, degp, W3)
    accp = agg_k(y, src3, dst3, zrows)
    y = _ep(accp, y, b3r, degp, W4)
    accp = agg_k(y, src3, dst3, zrows)
    return _pool(accp, y, b4r, degp, batch3)


# speedup vs baseline: 1.4054x; 1.0817x over previous
import jax
import jax.numpy as jnp
from jax import lax
from jax.experimental import pallas as pl
from jax.experimental.pallas import tpu as pltpu
from jax.experimental.pallas import tpu_sc as plsc

N = 10000
D = 128
G = 256
NC, NS = 2, 16
NW = NC * NS
K = 128
NPAD = 10112
RPT = NPAD // NS
ZCH = (RPT - 1) // K + 1
BM = 1000
GRID = N // BM

_MESH = dict(core_axis_name="c", subcore_axis_name="s", num_cores=NC,
             num_subcores=NS)


def _agg_body(y_hbm, src_hbm, dst_hbm, zrows_hbm, outp_hbm, sidx_v, didx_v,
              rows_v, acc_sh, sem):
    c = lax.axis_index("c")
    s = lax.axis_index("s")
    wid = c * NS + s
    r0 = s * RPT
    cpw = src_hbm.shape[1]
    pltpu.sync_copy(zrows_hbm, rows_v)
    for t in range(ZCH):
        n = min(K, RPT - t * K)
        pltpu.sync_copy(rows_v.at[pl.ds(0, n)],
                        acc_sh.at[pl.ds(r0 + t * K, n)])
    plsc.subcore_barrier()

    def step(j, carry):
        pltpu.sync_copy(src_hbm.at[wid, j], sidx_v)
        gather = pltpu.async_copy(y_hbm.at[sidx_v], rows_v, sem)
        pltpu.sync_copy(dst_hbm.at[wid, j], didx_v)
        gather.wait()
        pltpu.sync_copy(rows_v, acc_sh.at[didx_v], add=True)
        return carry

    lax.fori_loop(0, cpw, step, 0)
    plsc.subcore_barrier()
    for t in range(ZCH):
        n = min(K, RPT - t * K)
        pltpu.sync_copy(acc_sh.at[pl.ds(r0 + t * K, n)],
                        rows_v.at[pl.ds(0, n)])
        pltpu.sync_copy(rows_v.at[pl.ds(0, n)],
                        outp_hbm.at[c, pl.ds(r0 + t * K, n)])


def _make_agg(cpw):
    return pl.kernel(
        _agg_body,
        out_type=jax.ShapeDtypeStruct((NC, NPAD, D), jnp.float32),
        mesh=plsc.VectorSubcoreMesh(**_MESH),
        scratch_types=[
            pltpu.VMEM((K,), jnp.int32),
            pltpu.VMEM((K,), jnp.int32),
            pltpu.VMEM((K, D), jnp.float32),
            pltpu.VMEM_SHARED((NPAD, D), jnp.float32),
            pltpu.SemaphoreType.DMA,
        ],
    )


def _dinv(degp_ref):
    dg = degp_ref[0, :, 0:1] + degp_ref[1, :, 0:1] + 1.0
    return lax.rsqrt(dg)


def _mm1_body(x_ref, w_ref, degp_ref, y_ref):
    y_ref[...] = _dinv(degp_ref) * jnp.dot(
        x_ref[...], w_ref[...], preferred_element_type=jnp.float32)


def _ep_body(accp_ref, yprev_ref, b_ref, degp_ref, w_ref, ynext_ref):
    dinv = _dinv(degp_ref)
    s = accp_ref[0] + accp_ref[1]
    h = jnp.maximum(dinv * (s + yprev_ref[...]) + b_ref[...], 0.0)
    ynext_ref[...] = dinv * jnp.dot(
        h, w_ref[...], preferred_element_type=jnp.float32)


def _pool_body(accp_ref, y_ref, b_ref, degp_ref, batch_ref, out_ref, sums,
               cnts):
    i = pl.program_id(0)
    dinv = _dinv(degp_ref)
    s = accp_ref[0] + accp_ref[1]
    h = jnp.maximum(dinv * (s + y_ref[...]) + b_ref[...], 0.0)
    bvec = batch_ref[0]
    gid = lax.broadcasted_iota(jnp.int32, (G, BM), 0)
    oh = (bvec == gid).astype(jnp.float32)

    @pl.when(i == 0)
    def _():
        sums[...] = jnp.zeros_like(sums)
        cnts[...] = jnp.zeros_like(cnts)

    sums[...] += jnp.dot(oh, h, preferred_element_type=jnp.float32)
    cnts[...] = cnts[...] + jnp.sum(oh, axis=1, keepdims=True)

    @pl.when(i == GRID - 1)
    def _():
        out_ref[...] = sums[...] / jnp.maximum(cnts[...], 1.0)


_spec_rows = pl.BlockSpec((BM, D), lambda i: (i, 0))
_spec_w = pl.BlockSpec((D, D), lambda i: (0, 0))
_spec_b = pl.BlockSpec((1, D), lambda i: (0, 0))
_spec_degp = pl.BlockSpec((NC, BM, D), lambda i: (0, i, 0))
_spec_accp = pl.BlockSpec((NC, BM, D), lambda i: (0, i, 0))

_mm1 = pl.pallas_call(
    _mm1_body,
    grid=(GRID,),
    in_specs=[_spec_rows, _spec_w, _spec_degp],
    out_specs=_spec_rows,
    out_shape=jax.ShapeDtypeStruct((N, D), jnp.float32),
)

_ep = pl.pallas_call(
    _ep_body,
    grid=(GRID,),
    in_specs=[_spec_accp, _spec_rows, _spec_b, _spec_degp, _spec_w],
    out_specs=_spec_rows,
    out_shape=jax.ShapeDtypeStruct((N, D), jnp.float32),
)

_pool = pl.pallas_call(
    _pool_body,
    grid=(GRID,),
    in_specs=[_spec_accp, _spec_rows, _spec_b, _spec_degp,
              pl.BlockSpec((1, 1, BM), lambda i: (i, 0, 0))],
    out_specs=pl.BlockSpec((G, D), lambda i: (0, 0)),
    out_shape=jax.ShapeDtypeStruct((G, D), jnp.float32),
    scratch_shapes=[pltpu.VMEM((G, D), jnp.float32),
                    pltpu.VMEM((G, D), jnp.float32)],
)


def kernel(x, edge_index, batch, W1, b1, W2, b2, W3, b3, W4, b4):
    src = edge_index[0].astype(jnp.int32)
    dst = edge_index[1].astype(jnp.int32)
    e = src.shape[0]
    cpw = -(-e // (NW * K))
    epad = NW * K * cpw
    src3 = jnp.pad(src, (0, epad - e)).reshape(NW, cpw, K)
    dst3 = jnp.pad(dst, (0, epad - e),
                   constant_values=N).reshape(NW, cpw, K)
    ones_tbl = jnp.ones((NPAD, D), jnp.float32)
    zrows = jnp.zeros((K, D), jnp.float32)
    batch3 = batch.astype(jnp.int32).reshape(GRID, 1, BM)

    agg_k = _make_agg(cpw)

    degp = agg_k(ones_tbl, dst3, dst3, zrows)
    b1r, b2r, b3r, b4r = (b.reshape(1, D) for b in (b1, b2, b3, b4))

    y = _mm1(x, W1, degp)
    accp = agg_k(y, src3, dst3, zrows)
    y = _ep(accp, y, b1r, degp, W2)
    accp = agg_k(y, src3, dst3, zrows)
    y = _ep(accp, y, b2r, degp, W3)
    accp = agg_k(y, src3, dst3, zrows)
    y = _ep(accp, y, b3r, degp, W4)
    accp = agg_k(y, src3, dst3, zrows)
    return _pool(accp, y, b4r, degp, batch3)
